# baseline (speedup 1.0000x reference)
import functools

import jax
import jax.numpy as jnp
from jax import lax
from jax.experimental import pallas as pl
from jax.experimental.pallas import tpu as pltpu
from jax.experimental.pallas import tpu_sc as plsc

N_NODES = 10000
N_EDGES = 320000
D_FEAT = 128
D_AUG = 144

NC = 2
NS = 16
NW = NC * NS
E_W = N_EDGES // NW
K = 80
NCHUNK = E_W // K
U = 3
NBLK = NCHUNK // U
ZROWS = 80
NZCHUNK = N_NODES // ZROWS
ROWS_T = N_NODES // NS


def kernel(inputs, adj_indices, adj_values, W):
    adj_indices = adj_indices.astype(jnp.int32)
    src = adj_indices[1]
    dst = adj_indices[0]
    x_aug = jnp.concatenate(
        [inputs,
         jnp.ones((N_NODES, 1), jnp.float32),
         jnp.zeros((N_NODES, D_AUG - D_FEAT - 1), jnp.float32)], axis=1)
    vbits = jax.lax.bitcast_convert_type(adj_values, jnp.int32)
    packed = jnp.stack(
        [src.reshape(-1, K), dst.reshape(-1, K), vbits.reshape(-1, K)],
        axis=1)

    mesh = plsc.VectorSubcoreMesh(
        core_axis_name="c", subcore_axis_name="s",
        num_cores=NC, num_subcores=NS)

    @functools.partial(
        pl.kernel,
        mesh=mesh,
        out_type=jax.ShapeDtypeStruct((NC, N_NODES, D_AUG), jnp.float32),
        scratch_types=[
            pltpu.VMEM_SHARED((N_NODES, D_AUG), jnp.float32),
        ]
        + [pltpu.VMEM((3, K), jnp.int32) for _ in range(U)]
        + [pltpu.VMEM((K, D_AUG), jnp.float32) for _ in range(U)]
        + [pltpu.SemaphoreType.DMA for _ in range(3 * U)],
        compiler_params=pltpu.CompilerParams(
            use_tc_tiling_on_sc=False, needs_layout_passes=False),
    )
    def accumulate(pk_hbm, x_hbm, hid_out, hid_sh, *rest):
        pk_v = rest[0:U]
        rows_v = rest[U:2 * U]
        sem_idx = rest[2 * U:3 * U]
        sem_g = rest[3 * U:4 * U]
        sem_s = rest[4 * U:5 * U]
        cid = lax.axis_index("c")
        sid = lax.axis_index("s")
        wid = cid * NS + sid

        zero16 = jnp.zeros((16,), jnp.float32)

        def zrow(i, _):
            for c in range(D_AUG // 16):
                rows_v[0][i, pl.ds(c * 16, 16)] = zero16
            return 0
        lax.fori_loop(0, ZROWS, zrow, 0)

        nz = NZCHUNK // NS + 1
        for i in range(nz):
            zc = sid + i * NS

            @pl.when(zc < NZCHUNK)
            def _():
                pltpu.async_copy(rows_v[0],
                                 hid_sh.at[pl.ds(zc * ZROWS, ZROWS)],
                                 sem_g[i % 2])
        for i in range(nz):
            zc = sid + i * NS

            @pl.when(zc < NZCHUNK)
            def _():
                pltpu.make_async_copy(
                    rows_v[0],
                    hid_sh.at[pl.ds(zc * ZROWS, ZROWS)], sem_g[i % 2]).wait()

        plsc.subcore_barrier()

        g0 = wid * NCHUNK

        def idx_start(ci, u):
            pltpu.async_copy(pk_hbm.at[g0 + ci], pk_v[u], sem_idx[u])

        def idx_wait(u):
            pltpu.make_async_copy(pk_hbm.at[0], pk_v[u], sem_idx[u]).wait()

        def gather_start(u):
            pltpu.async_copy(x_hbm.at[pk_v[u].at[0]], rows_v[u], sem_g[u])

        def gather_wait(u):
            pltpu.make_async_copy(
                x_hbm.at[pk_v[u].at[0]], rows_v[u], sem_g[u]).wait()

        def scatter_start(u):
            pltpu.async_copy(rows_v[u], hid_sh.at[pk_v[u].at[1]], sem_s[u],
                             add=True)

        def scatter_wait(u):
            pltpu.make_async_copy(
                rows_v[u], hid_sh.at[pk_v[u].at[1]], sem_s[u]).wait()

        def compute(u):
            for i in range(K // 16):
                ev16 = jnp.exp(plsc.bitcast(
                    pk_v[u][2, pl.ds(i * 16, 16)], jnp.float32))
                for l in range(16):
                    j = i * 16 + l
                    ev = ev16[l]
                    for c in range(D_AUG // 16):
                        rows_v[u][j, pl.ds(c * 16, 16)] = (
                            rows_v[u][j, pl.ds(c * 16, 16)] * ev)

        idx_start(0, 0)
        idx_wait(0)
        gather_start(0)

        def step(ci, u):
            ci = jnp.int32(ci)
            un = (u + 1) % U

            @pl.when(ci >= U - 1)
            def _():
                scatter_wait(un)

            @pl.when(ci + 1 < NCHUNK)
            def _():
                idx_start(ci + 1, un)

            gather_wait(u)

            @pl.when(ci + 1 < NCHUNK)
            def _():
                idx_wait(un)
                gather_start(un)

            compute(u)
            scatter_start(u)

        def block_body(b, _):
            for u in range(U):
                step(b * U + u, u)
            return 0
        lax.fori_loop(0, NBLK, block_body, 0)
        for ci in range(NBLK * U, NCHUNK):
            step(ci, ci % U)

        for ci in range(NCHUNK - U + 1, NCHUNK):
            scatter_wait(ci % U)

        plsc.subcore_barrier()

        row0 = sid * ROWS_T
        pltpu.sync_copy(hid_sh.at[pl.ds(row0, ROWS_T)],
                        hid_out.at[cid, pl.ds(row0, ROWS_T)])

    hid_part = accumulate(packed, x_aug)

    BLK = 1000

    def combine_body(w_ref, h_ref, o_ref):
        h = h_ref[0, :, :D_FEAT] + h_ref[1, :, :D_FEAT]
        denom = h_ref[0, :, D_FEAT:D_FEAT + 1] + h_ref[1, :, D_FEAT:D_FEAT + 1]
        x = h * w_ref[...] / denom
        o_ref[...] = jnp.where(x > 0, x, jnp.exp(jnp.minimum(x, 0.0)) - 1.0)

    out = pl.pallas_call(
        combine_body,
        grid=(N_NODES // BLK,),
        in_specs=[
            pl.BlockSpec((1, D_FEAT), lambda i: (0, 0)),
            pl.BlockSpec((NC, BLK, D_AUG), lambda i: (0, i, 0)),
        ],
        out_specs=pl.BlockSpec((BLK, D_FEAT), lambda i: (i, 0)),
        out_shape=jax.ShapeDtypeStruct((N_NODES, D_FEAT), jnp.float32),
    )(W.reshape(1, D_FEAT), hid_part)
    return out


# speedup vs baseline: 1.3346x; 1.3346x over previous
import functools

import jax
import jax.numpy as jnp
from jax import lax
from jax.experimental import pallas as pl
from jax.experimental.pallas import tpu as pltpu
from jax.experimental.pallas import tpu_sc as plsc

N_NODES = 10000
N_EDGES = 320000
D_FEAT = 128

NC = 2
NS = 16
NW = NC * NS
E_W = N_EDGES // NW
K = 80
NCHUNK = E_W // K
U = 3
NBLK = NCHUNK // U
ZROWS = K
NZCHUNK = N_NODES // ZROWS
ROWS_T = N_NODES // NS


def kernel(inputs, adj_indices, adj_values, W):
    adj_indices = adj_indices.astype(jnp.int32)
    dst = adj_indices[0]
    src = adj_indices[1]
    vbits = jax.lax.bitcast_convert_type(adj_values, jnp.int32)
    packed = jnp.stack(
        [src.reshape(-1, K), dst.reshape(-1, K), vbits.reshape(-1, K)],
        axis=1)

    mesh = plsc.VectorSubcoreMesh(
        core_axis_name="c", subcore_axis_name="s",
        num_cores=NC, num_subcores=NS)

    @functools.partial(
        pl.kernel,
        mesh=mesh,
        out_type=[
            jax.ShapeDtypeStruct((NC, N_NODES, D_FEAT), jnp.float32),
            jax.ShapeDtypeStruct((NC, N_NODES, 16), jnp.float32),
        ],
        scratch_types=[
            pltpu.VMEM_SHARED((N_NODES, D_FEAT), jnp.float32),
            pltpu.VMEM_SHARED((N_NODES, 16), jnp.float32),
        ]
        + [pltpu.VMEM((3, K), jnp.int32) for _ in range(U)]
        + [pltpu.VMEM((K, D_FEAT), jnp.float32) for _ in range(U)]
        + [pltpu.VMEM((K, 16), jnp.float32) for _ in range(U)]
        + [pltpu.SemaphoreType.DMA for _ in range(3 * U)],
        compiler_params=pltpu.CompilerParams(
            use_tc_tiling_on_sc=False, needs_layout_passes=False),
    )
    def accumulate(pk_hbm, x_hbm, hid_out, rs_out, hid_sh, rs_sh, *rest):
        pk_v = rest[0:U]
        rows_v = rest[U:2 * U]
        rsb_v = rest[2 * U:3 * U]
        sem_idx = rest[3 * U:4 * U]
        sem_g = rest[4 * U:5 * U]
        sem_s = rest[5 * U:6 * U]
        cid = lax.axis_index("c")
        sid = lax.axis_index("s")
        wid = cid * NS + sid

        zero16 = jnp.zeros((16,), jnp.float32)

        def zrow(i, _):
            for u in range(U):
                for c in range(D_FEAT // 16):
                    rows_v[u][i, pl.ds(c * 16, 16)] = zero16
                rsb_v[u][i, :] = zero16
            return 0
        lax.fori_loop(0, K, zrow, 0)

        for i in range(NZCHUNK // NS + 1):
            zc = sid + i * NS

            @pl.when(zc < NZCHUNK)
            def _():
                pltpu.async_copy(rows_v[0],
                                 hid_sh.at[pl.ds(zc * ZROWS, ZROWS)],
                                 sem_g[i % U])
                pltpu.async_copy(rsb_v[0],
                                 rs_sh.at[pl.ds(zc * ZROWS, ZROWS)],
                                 sem_s[i % U])
        for i in range(NZCHUNK // NS + 1):
            zc = sid + i * NS

            @pl.when(zc < NZCHUNK)
            def _():
                pltpu.make_async_copy(
                    rows_v[0],
                    hid_sh.at[pl.ds(zc * ZROWS, ZROWS)], sem_g[i % U]).wait()
                pltpu.make_async_copy(
                    rsb_v[0],
                    rs_sh.at[pl.ds(zc * ZROWS, ZROWS)], sem_s[i % U]).wait()

        plsc.subcore_barrier()

        g0 = wid * NCHUNK

        def idx_start(ci, u):
            pltpu.async_copy(pk_hbm.at[g0 + ci], pk_v[u], sem_idx[u])

        def idx_wait(u):
            pltpu.make_async_copy(pk_hbm.at[0], pk_v[u], sem_idx[u]).wait()

        def gather_start(u):
            pltpu.async_copy(x_hbm.at[pk_v[u].at[0]], rows_v[u], sem_g[u])

        def gather_wait(u):
            pltpu.make_async_copy(
                x_hbm.at[pk_v[u].at[0]], rows_v[u], sem_g[u]).wait()

        def scatter_start(u):
            pltpu.async_copy(rows_v[u], hid_sh.at[pk_v[u].at[1]], sem_s[u],
                             add=True)
            pltpu.async_copy(rsb_v[u], rs_sh.at[pk_v[u].at[1]], sem_s[u],
                             add=True)

        def scatter_wait(u):
            pltpu.make_async_copy(
                rows_v[u], hid_sh.at[pk_v[u].at[1]], sem_s[u]).wait()
            pltpu.make_async_copy(
                rsb_v[u], rs_sh.at[pk_v[u].at[1]], sem_s[u]).wait()

        def compute(u):
            for i in range(K // 16):
                ev16 = jnp.exp(plsc.bitcast(
                    pk_v[u][2, pl.ds(i * 16, 16)], jnp.float32))
                pk_v[u][2, pl.ds(i * 16, 16)] = plsc.bitcast(ev16, jnp.int32)

            def scale_body(i, _):
                ev16 = plsc.bitcast(pk_v[u][2, pl.ds(i * 16, 16)], jnp.float32)
                for l in range(16):
                    j = i * 16 + l
                    ev = ev16[l]
                    rsb_v[u][j, :] = jnp.full((16,), ev, jnp.float32)
                    for c in range(D_FEAT // 16):
                        rows_v[u][j, pl.ds(c * 16, 16)] = (
                            rows_v[u][j, pl.ds(c * 16, 16)] * ev)
                return 0
            lax.fori_loop(0, K // 16, scale_body, 0)

        idx_start(0, 0)
        idx_wait(0)
        gather_start(0)

        def step(ci, u):
            ci = jnp.int32(ci)
            un = (u + 1) % U

            @pl.when(ci >= U - 1)
            def _():
                scatter_wait(un)

            @pl.when(ci + 1 < NCHUNK)
            def _():
                idx_start(ci + 1, un)

            gather_wait(u)

            @pl.when(ci + 1 < NCHUNK)
            def _():
                idx_wait(un)
                gather_start(un)

            compute(u)
            scatter_start(u)

        def block_body(b, _):
            for u in range(U):
                step(b * U + u, u)
            return 0
        lax.fori_loop(0, NBLK, block_body, 0)
        for ci in range(NBLK * U, NCHUNK):
            step(ci, ci % U)

        for ci in range(NCHUNK - U + 1, NCHUNK):
            scatter_wait(ci % U)

        plsc.subcore_barrier()

        row0 = sid * ROWS_T
        pltpu.sync_copy(hid_sh.at[pl.ds(row0, ROWS_T)],
                        hid_out.at[cid, pl.ds(row0, ROWS_T)])
        pltpu.sync_copy(rs_sh.at[pl.ds(row0, ROWS_T)],
                        rs_out.at[cid, pl.ds(row0, ROWS_T)])

    hid_part, rs_part = accumulate(packed, inputs)

    BLK = 1000

    def combine_body(w_ref, h_ref, rs_ref, o_ref):
        h = h_ref[0] + h_ref[1]
        denom = rs_ref[0, :, 0:1] + rs_ref[1, :, 0:1]
        x = h * w_ref[...] / denom
        o_ref[...] = jnp.where(x > 0, x, jnp.exp(jnp.minimum(x, 0.0)) - 1.0)

    out = pl.pallas_call(
        combine_body,
        grid=(N_NODES // BLK,),
        in_specs=[
            pl.BlockSpec((1, D_FEAT), lambda i: (0, 0)),
            pl.BlockSpec((NC, BLK, D_FEAT), lambda i: (0, i, 0)),
            pl.BlockSpec((NC, BLK, 16), lambda i: (0, i, 0)),
        ],
        out_specs=pl.BlockSpec((BLK, D_FEAT), lambda i: (i, 0)),
        out_shape=jax.ShapeDtypeStruct((N_NODES, D_FEAT), jnp.float32),
    )(W.reshape(1, D_FEAT), hid_part, rs_part)
    return out
